# Initial kernel scaffold; baseline (speedup 1.0000x reference)
import jax
import jax.numpy as jnp
from jax.experimental import pallas as pl


def kernel(features, pools):
    raise NotImplementedError("write your pallas kernel here")


# speedup vs baseline: 2.2523x; 2.2523x over previous
import functools

import jax
import jax.numpy as jnp
from jax import lax
from jax.experimental import pallas as pl
from jax.experimental.pallas import tpu as pltpu
from jax.experimental.pallas import tpu_sc as plsc

M = 50000
D = 256
K = 16
L = 16

NC, NS = 2, 16
NW = NC * NS
C_OUT = 8
B_PER_W = 1568
M_PAD = NW * B_PER_W
CHUNKS = B_PER_W // C_OUT


def _sc_max_pool(features, pools_flat):
    mesh = plsc.VectorSubcoreMesh(core_axis_name="c", subcore_axis_name="s")

    @functools.partial(
        pl.kernel,
        mesh=mesh,
        out_type=jax.ShapeDtypeStruct((M_PAD, D), jnp.float32),
        scratch_types=[
            pltpu.VMEM((C_OUT * K,), jnp.int32),
            pltpu.VMEM((C_OUT * K, D), jnp.float32),
            pltpu.VMEM((C_OUT, D), jnp.float32),
            pltpu.SemaphoreType.DMA,
        ],
    )
    def kern(feat_hbm, idx_hbm, out_hbm, idx_v, rows_v, out_v, sem):
        wid = lax.axis_index("s") * NC + lax.axis_index("c")

        def chunk_body(g, carry):
            base = wid * B_PER_W + g * C_OUT
            pltpu.sync_copy(idx_hbm.at[pl.ds(base * K, C_OUT * K)], idx_v)
            pltpu.async_copy(feat_hbm.at[idx_v], rows_v, sem).wait()

            def rj_body(rj, c2):
                r = rj // (D // L)
                j = rj - r * (D // L)
                row0 = r * K
                col = j * L
                acc = rows_v[row0, pl.ds(col, L)]
                for k in range(1, K):
                    acc = jnp.maximum(acc, rows_v[row0 + k, pl.ds(col, L)])
                out_v[r, pl.ds(col, L)] = acc
                return c2

            lax.fori_loop(0, C_OUT * (D // L), rj_body, 0)
            pltpu.sync_copy(out_v, out_hbm.at[pl.ds(base, C_OUT)])
            return carry

        lax.fori_loop(0, CHUNKS, chunk_body, 0)

    return kern(features, pools_flat)


@jax.jit
def kernel(features, pools):
    pools_flat = pools.astype(jnp.int32).reshape(-1)
    pad = M_PAD * K - pools_flat.shape[0]
    pools_flat = jnp.concatenate(
        [pools_flat, jnp.zeros((pad,), dtype=jnp.int32)]
    )
    out = _sc_max_pool(features, pools_flat)
    return out[:M]
